# baseline (speedup 1.0000x reference)
import functools

import jax
import jax.numpy as jnp
from jax import lax
from jax.experimental import pallas as pl
from jax.experimental.pallas import tpu as pltpu
from jax.experimental.pallas import tpu_sc as plsc

N_NODES = 50000
N_P = 50048
N_EDGES = 800000
N_H = 16
N_GRAPHS = 1024
G_P = 1040
NC = 2
NS = 16
NW = NC * NS
CH = 128
E_CHUNKS = N_EDGES // CH
G_CHUNKS = N_P // CH
NB = N_P // NS
E_BLK = 4000
E_GRID = N_EDGES // E_BLK

_SELU_SCALE = 1.0507009873554805
_SELU_ALPHA = 1.6732632423543772


_EMUL_EDGE = True
_EMUL_GRU = True
_EMUL_RO = True
_EMUL_FIN = True


def _dot(a, b, emul):
  if emul:
    a, b = _bf16r(a), _bf16r(b)
  return jnp.dot(a, b, preferred_element_type=jnp.float32,
                 precision=lax.Precision.HIGHEST)


def _bf16r(v):
  return v.astype(jnp.bfloat16).astype(jnp.float32)


def _selu(v):
  return _SELU_SCALE * jnp.where(
      v > 0, v, _SELU_ALPHA * (jnp.exp(jnp.minimum(v, 0.0)) - 1.0))


_mesh = lambda: plsc.VectorSubcoreMesh(
    core_axis_name="c", subcore_axis_name="s", num_cores=NC, num_subcores=NS)


def _sc_gather(h, idx2d):
  iters = (E_CHUNKS + NW - 1) // NW

  def body(h_ref, idx_ref, out_ref, idx_v, rows_v, sem):
    w = lax.axis_index("s") * NC + lax.axis_index("c")

    def step(it, carry):
      chunk = w + NW * it

      @pl.when(chunk < E_CHUNKS)
      def _():
        pltpu.sync_copy(idx_ref.at[chunk], idx_v)
        pltpu.async_copy(h_ref.at[idx_v], rows_v, sem).wait()
        pltpu.sync_copy(rows_v, out_ref.at[pl.ds(chunk * CH, CH)])

      return carry

    lax.fori_loop(0, iters, step, 0)

  fn = pl.kernel(
      body,
      out_type=jax.ShapeDtypeStruct((N_EDGES, N_H), jnp.float32),
      mesh=_mesh(),
      compiler_params=pltpu.CompilerParams(use_tc_tiling_on_sc=False),
      scratch_types=[
          pltpu.VMEM((CH,), jnp.int32),
          pltpu.VMEM((CH, N_H), jnp.float32),
          pltpu.SemaphoreType.DMA,
      ],
  )
  return fn(h, idx2d)


def _sc_scatter_nodes(m, idx2d, zeros_n):
  iters = (E_CHUNKS + NW - 1) // NW

  def body(m_ref, idx_ref, z_ref, out_ref, idx_v, rows_v, acc):
    c = lax.axis_index("c")
    s = lax.axis_index("s")
    w = s * NC + c
    r0 = s * NB
    pltpu.sync_copy(z_ref.at[pl.ds(r0, NB)], acc.at[pl.ds(r0, NB)])
    plsc.subcore_barrier()

    def step(it, carry):
      chunk = w + NW * it

      @pl.when(chunk < E_CHUNKS)
      def _():
        pltpu.sync_copy(idx_ref.at[chunk], idx_v)
        pltpu.sync_copy(m_ref.at[pl.ds(chunk * CH, CH)], rows_v)
        pltpu.sync_copy(rows_v, acc.at[idx_v], add=True)

      return carry

    lax.fori_loop(0, iters, step, 0)
    plsc.subcore_barrier()
    pltpu.sync_copy(acc.at[pl.ds(r0, NB)], out_ref.at[c].at[pl.ds(r0, NB)])

  fn = pl.kernel(
      body,
      out_type=jax.ShapeDtypeStruct((NC, N_P, N_H), jnp.float32),
      mesh=_mesh(),
      compiler_params=pltpu.CompilerParams(use_tc_tiling_on_sc=False),
      scratch_types=[
          pltpu.VMEM((CH,), jnp.int32),
          pltpu.VMEM((CH, N_H), jnp.float32),
          pltpu.VMEM_SHARED((N_P, N_H), jnp.float32),
      ],
  )
  return fn(m, idx2d, zeros_n)


def _sc_segsum_graphs(rr, seg2d, zeros_g):
  iters = (G_CHUNKS + NW - 1) // NW
  rows_per_tile = G_P // NS

  def body(rr_ref, idx_ref, z_ref, out_ref, idx_v, rows_v, acc):
    c = lax.axis_index("c")
    s = lax.axis_index("s")
    w = s * NC + c
    r0 = s * rows_per_tile
    pltpu.sync_copy(z_ref.at[pl.ds(r0, rows_per_tile)],
                    acc.at[pl.ds(r0, rows_per_tile)])
    plsc.subcore_barrier()

    def step(it, carry):
      chunk = w + NW * it

      @pl.when(chunk < G_CHUNKS)
      def _():
        pltpu.sync_copy(idx_ref.at[chunk], idx_v)
        pltpu.sync_copy(rr_ref.at[pl.ds(chunk * CH, CH)], rows_v)
        pltpu.sync_copy(rows_v, acc.at[idx_v], add=True)

      return carry

    lax.fori_loop(0, iters, step, 0)
    plsc.subcore_barrier()
    pltpu.sync_copy(acc.at[pl.ds(r0, rows_per_tile)],
                    out_ref.at[c].at[pl.ds(r0, rows_per_tile)])

  fn = pl.kernel(
      body,
      out_type=jax.ShapeDtypeStruct((NC, G_P, 128), jnp.float32),
      mesh=_mesh(),
      compiler_params=pltpu.CompilerParams(use_tc_tiling_on_sc=False),
      scratch_types=[
          pltpu.VMEM((CH,), jnp.int32),
          pltpu.VMEM((CH, 128), jnp.float32),
          pltpu.VMEM_SHARED((G_P, 128), jnp.float32),
      ],
  )
  return fn(rr, seg2d, zeros_g)


def _full(shape):
  return pl.BlockSpec(shape, lambda i: (0,) * len(shape))


def _tc_edge(e, hg, Wl1, bl1, Wl2, bl2, Wb1, bb1, Wb2, bb2):
  def body(e_ref, hg_ref, wl1, bl1_r, wl2, bl2_r, wb1, bb1_r, wb2, bb2_r,
           out_ref):
    ev = e_ref[...]
    u = _selu(ev * wl1[...] + bl1_r[...])
    ub = _selu(ev * wb1[...] + bb1_r[...])
    aT = _dot(u, wl2[...], _EMUL_EDGE) + bl2_r[...]
    hgv = hg_ref[...]
    acc = _dot(ub, wb2[...], _EMUL_EDGE) + bb2_r[...]
    for j in range(N_H):
      acc = acc + aT[:, N_H * j:N_H * (j + 1)] * hgv[:, j:j + 1]
    out_ref[...] = acc

  return pl.pallas_call(
      body,
      grid=(E_GRID,),
      in_specs=[
          pl.BlockSpec((E_BLK, 1), lambda i: (i, 0)),
          pl.BlockSpec((E_BLK, N_H), lambda i: (i, 0)),
          _full((1, 64)), _full((1, 64)),
          _full((64, 256)), _full((1, 256)),
          _full((1, 64)), _full((1, 64)),
          _full((64, N_H)), _full((1, N_H)),
      ],
      out_specs=pl.BlockSpec((E_BLK, N_H), lambda i: (i, 0)),
      out_shape=jax.ShapeDtypeStruct((N_EDGES, N_H), jnp.float32),
  )(e, hg, Wl1, bl1, Wl2, bl2, Wb1, bb1, Wb2, bb2)


def _tc_gru(h, p0, p1, Wgz, Wgr, Wgn, Ugz, Ugr, Ugn, bgz, bgr, bgn):
  def body(h_ref, p0_ref, p1_ref, wgz, wgr, wgn, ugz, ugr, ugn,
           bz, br_, bn, out_ref):
    hv = h_ref[...]
    m = p0_ref[...] + p1_ref[...]
    dot = lambda a, b: _dot(a, b, _EMUL_GRU)
    z = jax.nn.sigmoid(dot(m, wgz[...]) + bz[...] + dot(hv, ugz[...]))
    r = jax.nn.sigmoid(dot(m, wgr[...]) + br_[...] + dot(hv, ugr[...]))
    n = jnp.tanh(dot(m, wgn[...]) + bn[...] + r * dot(hv, ugn[...]))
    out_ref[...] = z * hv + (1.0 - z) * n

  blk = lambda i: (i, 0)
  return pl.pallas_call(
      body,
      grid=(NS,),
      in_specs=[pl.BlockSpec((NB, N_H), blk)] * 3
      + [_full((N_H, N_H))] * 6 + [_full((1, N_H))] * 3,
      out_specs=pl.BlockSpec((NB, N_H), blk),
      out_shape=jax.ShapeDtypeStruct((N_P, N_H), jnp.float32),
  )(h, p0, p1, Wgz, Wgr, Wgn, Ugz, Ugr, Ugn, bgz, bgr, bgn)


def _tc_readout(h, xp, Wi32, bi, Wr, br, Wj132, bj1, Wj2, bj2):
  def body(h_ref, x_ref, wi, bi_r, wr, br_r, wj1, bj1_r, wj2, bj2_r, out_ref):
    hv = h_ref[...]
    xv = x_ref[...]
    hx = jnp.concatenate(
        [hv, xv, jnp.zeros((hv.shape[0], 14), jnp.float32)], axis=1)
    dot = lambda a, b: _dot(a, b, _EMUL_RO)
    i_ = jnp.tanh(dot(hx, wi[...]) + bi_r[...])
    rr = jax.nn.sigmoid(dot(i_, wr[...]) + br_r[...])
    j = _selu(dot(hx, wj1[...]) + bj1_r[...])
    j = dot(j, wj2[...]) + bj2_r[...]
    out_ref[...] = rr * j

  blk = lambda i: (i, 0)
  return pl.pallas_call(
      body,
      grid=(NS,),
      in_specs=[
          pl.BlockSpec((NB, N_H), blk),
          pl.BlockSpec((NB, 2), blk),
          _full((32, 128)), _full((1, 128)),
          _full((128, 128)), _full((1, 128)),
          _full((32, 128)), _full((1, 128)),
          _full((128, 128)), _full((1, 128)),
      ],
      out_specs=pl.BlockSpec((NB, 128), blk),
      out_shape=jax.ShapeDtypeStruct((N_P, 128), jnp.float32),
  )(h, xp, Wi32, bi, Wr, br, Wj132, bj1, Wj2, bj2)


def _tc_final(nb0, nb1, Wf1, bf1, Wf2p, bf2):
  def body(n0_ref, n1_ref, wf1, bf1_r, wf2, bf2_r, out_ref):
    nb = n0_ref[...] + n1_ref[...]
    t = _selu(_dot(nb, wf1[...], _EMUL_FIN) + bf1_r[...])
    out_ref[...] = _dot(t, wf2[...], _EMUL_FIN) + bf2_r[0, 0]

  return pl.pallas_call(
      body,
      grid=(1,),
      in_specs=[
          _full((N_GRAPHS, 128)), _full((N_GRAPHS, 128)),
          _full((128, 128)), _full((1, 128)),
          _full((128, 128)), _full((1, 1)),
      ],
      out_specs=_full((N_GRAPHS, 128)),
      out_shape=jax.ShapeDtypeStruct((N_GRAPHS, 128), jnp.float32),
  )(nb0, nb1, Wf1, bf1, Wf2p, bf2)


def kernel(x, e, first, second, segment, Wl1, bl1, Wl2, bl2, Wb1, bb1,
           Wb2, bb2, Wg, Ug, bg, Wi, bi, Wr, br, Wj1, bj1, Wj2, bj2,
           Wf1, bf1, Wf2, bf2):
  pad_n = N_P - N_NODES
  h = jnp.pad(x, ((0, pad_n), (0, N_H - 2)))
  xp = jnp.pad(x, ((0, pad_n), (0, 0)))
  first2d = first.reshape(E_CHUNKS, CH)
  second2d = second.reshape(E_CHUNKS, CH)
  seg_pad = N_GRAPHS + (jnp.arange(pad_n, dtype=jnp.int32) % (G_P - N_GRAPHS))
  seg2d = jnp.concatenate([segment, seg_pad]).reshape(G_CHUNKS, CH)
  zeros_n = jnp.zeros((N_P, N_H), jnp.float32)
  zeros_g = jnp.zeros((G_P, 128), jnp.float32)

  Wl2T = Wl2.reshape(64, N_H, N_H).transpose(0, 2, 1).reshape(64, 256)
  bl2T = bl2.reshape(N_H, N_H).T.reshape(256)
  Wl1r = Wl1.reshape(1, 64)
  Wb1r = Wb1.reshape(1, 64)
  bl1r = bl1.reshape(1, 64)
  bb1r = bb1.reshape(1, 64)
  bl2r = bl2T.reshape(1, 256)
  bb2r = bb2.reshape(1, N_H)
  Wgz, Wgr, Wgn = Wg[:, :16], Wg[:, 16:32], Wg[:, 32:]
  Ugz, Ugr, Ugn = Ug[:, :16], Ug[:, 16:32], Ug[:, 32:]
  bgz, bgr, bgn = bg[:16].reshape(1, 16), bg[16:32].reshape(1, 16), \
      bg[32:].reshape(1, 16)
  Wi32 = jnp.pad(Wi, ((0, 14), (0, 0)))
  Wj132 = jnp.pad(Wj1, ((0, 14), (0, 0)))
  bir = bi.reshape(1, 128)
  brr = br.reshape(1, 128)
  bj1r = bj1.reshape(1, 128)
  bj2r = bj2.reshape(1, 128)
  bf1r = bf1.reshape(1, 128)
  Wf2p = jnp.pad(Wf2, ((0, 0), (0, 127)))
  bf2r = bf2.reshape(1, 1)

  for _ in range(2):
    hg = _sc_gather(h, first2d)
    m = _tc_edge(e, hg, Wl1r, bl1r, Wl2T, bl2r, Wb1r, bb1r, Wb2, bb2r)
    parts = _sc_scatter_nodes(m, second2d, zeros_n)
    h = _tc_gru(h, parts[0], parts[1], Wgz, Wgr, Wgn, Ugz, Ugr, Ugn,
                bgz, bgr, bgn)

  rr = _tc_readout(h, xp, Wi32, bir, Wr, brr, Wj132, bj1r, Wj2, bj2r)
  partsg = _sc_segsum_graphs(rr, seg2d, zeros_g)
  f = _tc_final(partsg[0, :N_GRAPHS], partsg[1, :N_GRAPHS], Wf1, bf1r,
                Wf2p, bf2r)
  return f[:, :1]


# speedup vs baseline: 1.1061x; 1.1061x over previous
import functools

import jax
import jax.numpy as jnp
from jax import lax
from jax.experimental import pallas as pl
from jax.experimental.pallas import tpu as pltpu
from jax.experimental.pallas import tpu_sc as plsc

N_NODES = 50000
N_P = 50048
N_EDGES = 800000
N_H = 16
N_GRAPHS = 1024
G_P = 1040
NC = 2
NS = 16
NW = NC * NS
CH = 128
E_CHUNKS = N_EDGES // CH
G_CHUNKS = N_P // CH
NB = N_P // NS
E_BLK = 4000
E_GRID = N_EDGES // E_BLK

_SELU_SCALE = 1.0507009873554805
_SELU_ALPHA = 1.6732632423543772


_EMUL_EDGE = True
_EMUL_GRU = True
_EMUL_RO = True
_EMUL_FIN = True


def _dot(a, b, emul):
  if emul:
    return jnp.dot(a.astype(jnp.bfloat16), b.astype(jnp.bfloat16),
                   preferred_element_type=jnp.float32)
  return jnp.dot(a, b, preferred_element_type=jnp.float32,
                 precision=lax.Precision.HIGHEST)


def _bf16r(v):
  return v.astype(jnp.bfloat16).astype(jnp.float32)


def _selu(v):
  return _SELU_SCALE * jnp.where(
      v > 0, v, _SELU_ALPHA * (jnp.exp(jnp.minimum(v, 0.0)) - 1.0))


_mesh = lambda: plsc.VectorSubcoreMesh(
    core_axis_name="c", subcore_axis_name="s", num_cores=NC, num_subcores=NS)


def _sc_gather(h, idx2d):
  iters = (E_CHUNKS + NW - 1) // NW

  def body(h_ref, idx_ref, out_ref, idx_v, rows_v, sem):
    w = lax.axis_index("s") * NC + lax.axis_index("c")

    def step(it, carry):
      chunk = w + NW * it

      @pl.when(chunk < E_CHUNKS)
      def _():
        pltpu.sync_copy(idx_ref.at[chunk], idx_v)
        pltpu.async_copy(h_ref.at[idx_v], rows_v, sem).wait()
        pltpu.sync_copy(rows_v, out_ref.at[pl.ds(chunk * CH, CH)])

      return carry

    lax.fori_loop(0, iters, step, 0)

  fn = pl.kernel(
      body,
      out_type=jax.ShapeDtypeStruct((N_EDGES, N_H), jnp.float32),
      mesh=_mesh(),
      compiler_params=pltpu.CompilerParams(use_tc_tiling_on_sc=False),
      scratch_types=[
          pltpu.VMEM((CH,), jnp.int32),
          pltpu.VMEM((CH, N_H), jnp.float32),
          pltpu.SemaphoreType.DMA,
      ],
  )
  return fn(h, idx2d)


def _sc_scatter_nodes(m, idx2d, zeros_n):
  iters = (E_CHUNKS + NW - 1) // NW

  def body(m_ref, idx_ref, z_ref, out_ref, idx_v, rows_v, acc):
    c = lax.axis_index("c")
    s = lax.axis_index("s")
    w = s * NC + c
    r0 = s * NB
    pltpu.sync_copy(z_ref.at[pl.ds(r0, NB)], acc.at[pl.ds(r0, NB)])
    plsc.subcore_barrier()

    def step(it, carry):
      chunk = w + NW * it

      @pl.when(chunk < E_CHUNKS)
      def _():
        pltpu.sync_copy(idx_ref.at[chunk], idx_v)
        pltpu.sync_copy(m_ref.at[pl.ds(chunk * CH, CH)], rows_v)
        pltpu.sync_copy(rows_v, acc.at[idx_v], add=True)

      return carry

    lax.fori_loop(0, iters, step, 0)
    plsc.subcore_barrier()
    pltpu.sync_copy(acc.at[pl.ds(r0, NB)], out_ref.at[c].at[pl.ds(r0, NB)])

  fn = pl.kernel(
      body,
      out_type=jax.ShapeDtypeStruct((NC, N_P, N_H), jnp.float32),
      mesh=_mesh(),
      compiler_params=pltpu.CompilerParams(use_tc_tiling_on_sc=False),
      scratch_types=[
          pltpu.VMEM((CH,), jnp.int32),
          pltpu.VMEM((CH, N_H), jnp.float32),
          pltpu.VMEM_SHARED((N_P, N_H), jnp.float32),
      ],
  )
  return fn(m, idx2d, zeros_n)


def _sc_segsum_graphs(rr, seg2d, zeros_g):
  iters = (G_CHUNKS + NW - 1) // NW
  rows_per_tile = G_P // NS

  def body(rr_ref, idx_ref, z_ref, out_ref, idx_v, rows_v, acc):
    c = lax.axis_index("c")
    s = lax.axis_index("s")
    w = s * NC + c
    r0 = s * rows_per_tile
    pltpu.sync_copy(z_ref.at[pl.ds(r0, rows_per_tile)],
                    acc.at[pl.ds(r0, rows_per_tile)])
    plsc.subcore_barrier()

    def step(it, carry):
      chunk = w + NW * it

      @pl.when(chunk < G_CHUNKS)
      def _():
        pltpu.sync_copy(idx_ref.at[chunk], idx_v)
        pltpu.sync_copy(rr_ref.at[pl.ds(chunk * CH, CH)], rows_v)
        pltpu.sync_copy(rows_v, acc.at[idx_v], add=True)

      return carry

    lax.fori_loop(0, iters, step, 0)
    plsc.subcore_barrier()
    pltpu.sync_copy(acc.at[pl.ds(r0, rows_per_tile)],
                    out_ref.at[c].at[pl.ds(r0, rows_per_tile)])

  fn = pl.kernel(
      body,
      out_type=jax.ShapeDtypeStruct((NC, G_P, 128), jnp.float32),
      mesh=_mesh(),
      compiler_params=pltpu.CompilerParams(use_tc_tiling_on_sc=False),
      scratch_types=[
          pltpu.VMEM((CH,), jnp.int32),
          pltpu.VMEM((CH, 128), jnp.float32),
          pltpu.VMEM_SHARED((G_P, 128), jnp.float32),
      ],
  )
  return fn(rr, seg2d, zeros_g)


def _full(shape):
  return pl.BlockSpec(shape, lambda i: (0,) * len(shape))


def _tc_edge(e, hg, Wl1, bl1, Wl2, bl2, Wb1, bb1, Wb2, bb2):
  def body(e_ref, hg_ref, wl1, bl1_r, wl2, bl2_r, wb1, bb1_r, wb2, bb2_r,
           out_ref):
    ev = e_ref[...]
    u = _selu(ev * wl1[...] + bl1_r[...])
    ub = _selu(ev * wb1[...] + bb1_r[...])
    aT = _dot(u, wl2[...], _EMUL_EDGE) + bl2_r[...]
    hgv = hg_ref[...]
    acc = _dot(ub, wb2[...], _EMUL_EDGE) + bb2_r[...]
    for j in range(N_H):
      acc = acc + aT[:, N_H * j:N_H * (j + 1)] * hgv[:, j:j + 1]
    out_ref[...] = acc

  return pl.pallas_call(
      body,
      grid=(E_GRID,),
      in_specs=[
          pl.BlockSpec((E_BLK, 1), lambda i: (i, 0)),
          pl.BlockSpec((E_BLK, N_H), lambda i: (i, 0)),
          _full((1, 64)), _full((1, 64)),
          _full((64, 256)), _full((1, 256)),
          _full((1, 64)), _full((1, 64)),
          _full((64, N_H)), _full((1, N_H)),
      ],
      out_specs=pl.BlockSpec((E_BLK, N_H), lambda i: (i, 0)),
      out_shape=jax.ShapeDtypeStruct((N_EDGES, N_H), jnp.float32),
  )(e, hg, Wl1, bl1, Wl2, bl2, Wb1, bb1, Wb2, bb2)


def _tc_gru(h, p0, p1, Wgz, Wgr, Wgn, Ugz, Ugr, Ugn, bgz, bgr, bgn):
  def body(h_ref, p0_ref, p1_ref, wgz, wgr, wgn, ugz, ugr, ugn,
           bz, br_, bn, out_ref):
    hv = h_ref[...]
    m = p0_ref[...] + p1_ref[...]
    dot = lambda a, b: _dot(a, b, _EMUL_GRU)
    z = jax.nn.sigmoid(dot(m, wgz[...]) + bz[...] + dot(hv, ugz[...]))
    r = jax.nn.sigmoid(dot(m, wgr[...]) + br_[...] + dot(hv, ugr[...]))
    n = jnp.tanh(dot(m, wgn[...]) + bn[...] + r * dot(hv, ugn[...]))
    out_ref[...] = z * hv + (1.0 - z) * n

  blk = lambda i: (i, 0)
  return pl.pallas_call(
      body,
      grid=(NS,),
      in_specs=[pl.BlockSpec((NB, N_H), blk)] * 3
      + [_full((N_H, N_H))] * 6 + [_full((1, N_H))] * 3,
      out_specs=pl.BlockSpec((NB, N_H), blk),
      out_shape=jax.ShapeDtypeStruct((N_P, N_H), jnp.float32),
  )(h, p0, p1, Wgz, Wgr, Wgn, Ugz, Ugr, Ugn, bgz, bgr, bgn)


def _tc_readout(h, xp, Wi32, bi, Wr, br, Wj132, bj1, Wj2, bj2):
  def body(h_ref, x_ref, wi, bi_r, wr, br_r, wj1, bj1_r, wj2, bj2_r, out_ref):
    hv = h_ref[...]
    xv = x_ref[...]
    hx = jnp.concatenate(
        [hv, xv, jnp.zeros((hv.shape[0], 14), jnp.float32)], axis=1)
    dot = lambda a, b: _dot(a, b, _EMUL_RO)
    i_ = jnp.tanh(dot(hx, wi[...]) + bi_r[...])
    rr = jax.nn.sigmoid(dot(i_, wr[...]) + br_r[...])
    j = _selu(dot(hx, wj1[...]) + bj1_r[...])
    j = dot(j, wj2[...]) + bj2_r[...]
    out_ref[...] = rr * j

  blk = lambda i: (i, 0)
  return pl.pallas_call(
      body,
      grid=(NS,),
      in_specs=[
          pl.BlockSpec((NB, N_H), blk),
          pl.BlockSpec((NB, 2), blk),
          _full((32, 128)), _full((1, 128)),
          _full((128, 128)), _full((1, 128)),
          _full((32, 128)), _full((1, 128)),
          _full((128, 128)), _full((1, 128)),
      ],
      out_specs=pl.BlockSpec((NB, 128), blk),
      out_shape=jax.ShapeDtypeStruct((N_P, 128), jnp.float32),
  )(h, xp, Wi32, bi, Wr, br, Wj132, bj1, Wj2, bj2)


def _tc_final(nb0, nb1, Wf1, bf1, Wf2p, bf2):
  def body(n0_ref, n1_ref, wf1, bf1_r, wf2, bf2_r, out_ref):
    nb = n0_ref[...] + n1_ref[...]
    t = _selu(_dot(nb, wf1[...], _EMUL_FIN) + bf1_r[...])
    out_ref[...] = _dot(t, wf2[...], _EMUL_FIN) + bf2_r[0, 0]

  return pl.pallas_call(
      body,
      grid=(1,),
      in_specs=[
          _full((N_GRAPHS, 128)), _full((N_GRAPHS, 128)),
          _full((128, 128)), _full((1, 128)),
          _full((128, 128)), _full((1, 1)),
      ],
      out_specs=_full((N_GRAPHS, 128)),
      out_shape=jax.ShapeDtypeStruct((N_GRAPHS, 128), jnp.float32),
  )(nb0, nb1, Wf1, bf1, Wf2p, bf2)


def kernel(x, e, first, second, segment, Wl1, bl1, Wl2, bl2, Wb1, bb1,
           Wb2, bb2, Wg, Ug, bg, Wi, bi, Wr, br, Wj1, bj1, Wj2, bj2,
           Wf1, bf1, Wf2, bf2):
  pad_n = N_P - N_NODES
  h = jnp.pad(x, ((0, pad_n), (0, N_H - 2)))
  xp = jnp.pad(x, ((0, pad_n), (0, 0)))
  first2d = first.reshape(E_CHUNKS, CH)
  second2d = second.reshape(E_CHUNKS, CH)
  seg_pad = N_GRAPHS + (jnp.arange(pad_n, dtype=jnp.int32) % (G_P - N_GRAPHS))
  seg2d = jnp.concatenate([segment, seg_pad]).reshape(G_CHUNKS, CH)
  zeros_n = jnp.zeros((N_P, N_H), jnp.float32)
  zeros_g = jnp.zeros((G_P, 128), jnp.float32)

  Wl2T = Wl2.reshape(64, N_H, N_H).transpose(0, 2, 1).reshape(64, 256)
  bl2T = bl2.reshape(N_H, N_H).T.reshape(256)
  Wl1r = Wl1.reshape(1, 64)
  Wb1r = Wb1.reshape(1, 64)
  bl1r = bl1.reshape(1, 64)
  bb1r = bb1.reshape(1, 64)
  bl2r = bl2T.reshape(1, 256)
  bb2r = bb2.reshape(1, N_H)
  Wgz, Wgr, Wgn = Wg[:, :16], Wg[:, 16:32], Wg[:, 32:]
  Ugz, Ugr, Ugn = Ug[:, :16], Ug[:, 16:32], Ug[:, 32:]
  bgz, bgr, bgn = bg[:16].reshape(1, 16), bg[16:32].reshape(1, 16), \
      bg[32:].reshape(1, 16)
  Wi32 = jnp.pad(Wi, ((0, 14), (0, 0)))
  Wj132 = jnp.pad(Wj1, ((0, 14), (0, 0)))
  bir = bi.reshape(1, 128)
  brr = br.reshape(1, 128)
  bj1r = bj1.reshape(1, 128)
  bj2r = bj2.reshape(1, 128)
  bf1r = bf1.reshape(1, 128)
  Wf2p = jnp.pad(Wf2, ((0, 0), (0, 127)))
  bf2r = bf2.reshape(1, 1)

  for _ in range(2):
    hg = _sc_gather(h, first2d)
    m = _tc_edge(e, hg, Wl1r, bl1r, Wl2T, bl2r, Wb1r, bb1r, Wb2, bb2r)
    parts = _sc_scatter_nodes(m, second2d, zeros_n)
    h = _tc_gru(h, parts[0], parts[1], Wgz, Wgr, Wgn, Ugz, Ugr, Ugn,
                bgz, bgr, bgn)

  rr = _tc_readout(h, xp, Wi32, bir, Wr, brr, Wj132, bj1r, Wj2, bj2r)
  partsg = _sc_segsum_graphs(rr, seg2d, zeros_g)
  f = _tc_final(partsg[0, :N_GRAPHS], partsg[1, :N_GRAPHS], Wf1, bf1r,
                Wf2p, bf2r)
  return f[:, :1]
